# baseline (speedup 1.0000x reference)
import functools

import jax
import jax.numpy as jnp
import numpy as np
from jax.experimental import pallas as pl

LAST_Q = 64
V_TOPK = 256
S_TOPK = 512
BQ = 128
NEG = -1e9


def _pattern_indices(q, k):
    B, H, S, D = q.shape
    scale = 1.0 / np.sqrt(D)
    qe = q[:, :, -LAST_Q:, :]
    est = jnp.einsum('bhqd,bhkd->bhqk', qe, k) * scale
    rows = jnp.arange(S - LAST_Q, S)[:, None]
    cols = jnp.arange(S)[None, :]
    causal_e = cols <= rows
    est = jnp.where(causal_e[None, None], est, -jnp.inf)
    est = jax.nn.softmax(est, axis=-1)

    vert = est.sum(axis=2)
    _, v_idx = jax.lax.top_k(vert, V_TOPK)

    est_rev = jnp.where(causal_e[None, None], est, 0.0)[..., ::-1]
    w = S + LAST_Q
    padded = jnp.pad(est_rev, ((0, 0), (0, 0), (0, 0), (0, LAST_Q)))
    flat = padded.reshape(B, H, LAST_Q * w)[:, :, :LAST_Q * (w - 1)]
    shifted = flat.reshape(B, H, LAST_Q, w - 1)
    slash = shifted.sum(axis=2)[..., LAST_Q - 1:LAST_Q - 1 + S]
    _, s_idx = jax.lax.top_k(slash, S_TOPK)
    return v_idx, s_idx


def _attn_body(q_ref, k_ref, v_ref, vb_ref, u_ref, o_ref, *, scale, nt):
    i = pl.program_id(1)
    S = k_ref.shape[1]
    qb = (q_ref[0] * scale).astype(jnp.bfloat16)
    kb = k_ref[0]
    s_t = jax.lax.dot_general(kb, qb, (((1,), (1,)), ((), ())),
                              preferred_element_type=jnp.float32)
    u = u_ref[0, pl.ds((nt - 1 - i) * BQ, S), :]
    bias = jnp.maximum(u, vb_ref[0])
    col_id = jax.lax.broadcasted_iota(jnp.int32, (S, BQ), 0)
    row_id = i * BQ + jax.lax.broadcasted_iota(jnp.int32, (S, BQ), 1)
    sm = jnp.where(col_id <= row_id, s_t + bias, NEG)
    m = jnp.max(sm, axis=0, keepdims=True)
    p = jnp.exp(sm - m)
    l = jnp.sum(p, axis=0, keepdims=True)
    pn = (p * (1.0 / l)).astype(jnp.bfloat16)
    acc = jax.lax.dot_general(pn, v_ref[0],
                              (((0,), (0,)), ((), ())),
                              preferred_element_type=jnp.float32)
    o_ref[0] = acc


def _attn(qh, kh, vh, v_bias_t, u_t, *, interpret=False):
    H, S, D = qh.shape
    nt = S // BQ
    scale = 1.0 / np.sqrt(D)
    kb = kh.astype(jnp.bfloat16)
    vb16 = vh.astype(jnp.bfloat16)
    body = functools.partial(_attn_body, scale=scale, nt=nt)
    return pl.pallas_call(
        body,
        grid=(H, nt),
        in_specs=[
            pl.BlockSpec((1, BQ, D), lambda h, i: (h, i, 0)),
            pl.BlockSpec((1, S, D), lambda h, i: (h, 0, 0)),
            pl.BlockSpec((1, S, D), lambda h, i: (h, 0, 0)),
            pl.BlockSpec((1, S, 128), lambda h, i: (h, 0, 0)),
            pl.BlockSpec((1, S + BQ * nt, BQ), lambda h, i: (h, 0, 0)),
        ],
        out_specs=pl.BlockSpec((1, BQ, D), lambda h, i: (h, i, 0)),
        out_shape=jax.ShapeDtypeStruct((H, S, D), jnp.float32),
        interpret=interpret,
    )(qh, kb, vb16, v_bias_t, u_t)


def kernel(q, k, v):
    B, H, S, D = q.shape
    v_idx, s_idx = _pattern_indices(q, k)

    s_r = jnp.arange(S)
    v_bias = jnp.where((v_idx[..., None] == s_r).any(axis=-2), 0.0, NEG)
    s_bias = jnp.where((s_idx[..., None] == s_r).any(axis=-2), 0.0, NEG)

    nt = S // BQ
    f = jnp.concatenate(
        [s_bias[0][:, ::-1],
         jnp.full((H, S + BQ - 1 + (BQ - 1)), NEG, s_bias.dtype)], axis=1)
    u_t = jnp.stack([f[:, BQ - 1 - r:BQ - 1 - r + 2 * S]
                     for r in range(BQ)], axis=-1)

    v_bias_t = jnp.broadcast_to(v_bias[0][:, :, None], (H, S, 128))

    out = _attn(q[0], k[0], v[0], v_bias_t, u_t)
    out = out[None]

    qmin = jnp.minimum(v_idx.min(-1), s_idx.min(-1))
    mean_v = jnp.mean(v, axis=2, keepdims=True)
    empty = s_r[None, None, :, None] < qmin[:, :, None, None]
    return jnp.where(empty, mean_v, out)


# speedup vs baseline: 9.1813x; 9.1813x over previous
import functools

import jax
import jax.numpy as jnp
import numpy as np
from jax.experimental import pallas as pl

LAST_Q = 64
V_TOPK = 256
S_TOPK = 512
BQ = 128
NEG = -1e9


def _pattern_indices(q, k):
    B, H, S, D = q.shape
    scale = 1.0 / np.sqrt(D)
    qe = q[:, :, -LAST_Q:, :]
    est = jnp.einsum('bhqd,bhkd->bhqk', qe, k) * scale
    rows = jnp.arange(S - LAST_Q, S)[:, None]
    cols = jnp.arange(S)[None, :]
    causal_e = cols <= rows
    est = jnp.where(causal_e[None, None], est, -jnp.inf)
    est = jax.nn.softmax(est, axis=-1)

    vert = est.sum(axis=2)
    _, v_idx = jax.lax.top_k(vert, V_TOPK)

    est_rev = jnp.where(causal_e[None, None], est, 0.0)[..., ::-1]
    w = S + LAST_Q
    padded = jnp.pad(est_rev, ((0, 0), (0, 0), (0, 0), (0, LAST_Q)))
    flat = padded.reshape(B, H, LAST_Q * w)[:, :, :LAST_Q * (w - 1)]
    shifted = flat.reshape(B, H, LAST_Q, w - 1)
    slash = shifted.sum(axis=2)[..., LAST_Q - 1:LAST_Q - 1 + S]
    _, s_idx = jax.lax.top_k(slash, S_TOPK)
    return v_idx, s_idx


def _attn_body(q_ref, k_ref, v_ref, vb_ref, u_ref, o_ref, *, scale, nt):
    i = pl.program_id(1)
    S = k_ref.shape[1]
    qb = (q_ref[0] * scale).astype(jnp.bfloat16)
    kb = k_ref[0]
    s = jax.lax.dot_general(qb, kb, (((1,), (1,)), ((), ())),
                            preferred_element_type=jnp.float32)
    uq = u_ref[0, :, pl.ds((nt - 1 - i) * BQ, S)]
    bias = jnp.maximum(uq, vb_ref[0])
    row_id = i * BQ + jax.lax.broadcasted_iota(jnp.int32, (BQ, S), 0)
    col_id = jax.lax.broadcasted_iota(jnp.int32, (BQ, S), 1)
    sm = jnp.where(col_id <= row_id, s + bias, NEG)
    m = jnp.max(sm, axis=1, keepdims=True)
    p = jnp.exp(sm - m)
    l = jnp.sum(p, axis=1, keepdims=True)
    pn = (p * (1.0 / l)).astype(jnp.bfloat16)
    acc = jax.lax.dot_general(pn, v_ref[0],
                              (((1,), (0,)), ((), ())),
                              preferred_element_type=jnp.float32)
    o_ref[0] = acc


def _attn(qh, kh, vh, v_bias_r, u_q, *, interpret=False):
    H, S, D = qh.shape
    nt = S // BQ
    scale = 1.0 / np.sqrt(D)
    kb = kh.astype(jnp.bfloat16)
    vb16 = vh.astype(jnp.bfloat16)
    body = functools.partial(_attn_body, scale=scale, nt=nt)
    return pl.pallas_call(
        body,
        grid=(H, nt),
        in_specs=[
            pl.BlockSpec((1, BQ, D), lambda h, i: (h, i, 0)),
            pl.BlockSpec((1, S, D), lambda h, i: (h, 0, 0)),
            pl.BlockSpec((1, S, D), lambda h, i: (h, 0, 0)),
            pl.BlockSpec((1, 1, S), lambda h, i: (h, 0, 0)),
            pl.BlockSpec((1, BQ, 2 * S), lambda h, i: (h, 0, 0)),
        ],
        out_specs=pl.BlockSpec((1, BQ, D), lambda h, i: (h, i, 0)),
        out_shape=jax.ShapeDtypeStruct((H, S, D), jnp.float32),
        interpret=interpret,
    )(qh, kb, vb16, v_bias_r, u_q)


def kernel(q, k, v):
    B, H, S, D = q.shape
    v_idx, s_idx = _pattern_indices(q, k)

    s_r = jnp.arange(S)
    v_bias = jnp.where((v_idx[..., None] == s_r).any(axis=-2), 0.0, NEG)
    s_bias = jnp.where((s_idx[..., None] == s_r).any(axis=-2), 0.0, NEG)

    nt = S // BQ
    w2 = 2 * S + 2 * BQ
    h_vec = jnp.concatenate(
        [s_bias[0][:, ::-1], jnp.full((H, w2 - S), NEG, s_bias.dtype)], axis=1)
    m_all = jnp.broadcast_to(h_vec[:, None, :], (H, BQ, w2))
    skew = (m_all.reshape(H, BQ * w2)[:, :BQ * (w2 - 1)]
            .reshape(H, BQ, w2 - 1))
    u_q = skew[:, :, BQ - 1:BQ - 1 + 2 * S]

    out = _attn(q[0], k[0], v[0], v_bias[0][:, None, :], u_q)
    out = out[None]

    qmin = jnp.minimum(v_idx.min(-1), s_idx.min(-1))
    mean_v = jnp.mean(v, axis=2, keepdims=True)
    empty = s_r[None, None, :, None] < qmin[:, :, None, None]
    return jnp.where(empty, mean_v, out)


# speedup vs baseline: 10.3908x; 1.1317x over previous
import functools

import jax
import jax.numpy as jnp
import numpy as np
from jax.experimental import pallas as pl

LAST_Q = 64
V_TOPK = 256
S_TOPK = 512
BQ = 128
NEG = -1e9


def _pattern_indices(q, k):
    B, H, S, D = q.shape
    scale = 1.0 / np.sqrt(D)
    qe = q[:, :, -LAST_Q:, :]
    est = jnp.einsum('bhqd,bhkd->bhqk', qe, k) * scale
    rows = jnp.arange(S - LAST_Q, S)[:, None]
    cols = jnp.arange(S)[None, :]
    causal_e = cols <= rows
    est = jnp.where(causal_e[None, None], est, -jnp.inf)
    est = jax.nn.softmax(est, axis=-1)

    vert = est.sum(axis=2)
    _, v_idx = jax.lax.top_k(vert, V_TOPK)

    est_rev = jnp.where(causal_e[None, None], est, 0.0)[..., ::-1]
    w = S + LAST_Q
    padded = jnp.pad(est_rev, ((0, 0), (0, 0), (0, 0), (0, LAST_Q)))
    flat = padded.reshape(B, H, LAST_Q * w)[:, :, :LAST_Q * (w - 1)]
    shifted = flat.reshape(B, H, LAST_Q, w - 1)
    slash = shifted.sum(axis=2)[..., LAST_Q - 1:LAST_Q - 1 + S]
    _, s_idx = jax.lax.top_k(slash, S_TOPK)
    return v_idx, s_idx


def _attn_body(q_ref, k_ref, v_ref, vb_ref, u_ref, o_ref, *, scale, nt, i0):
    i = i0 + pl.program_id(1)
    SK = k_ref.shape[1]
    qb = (q_ref[0] * scale).astype(jnp.bfloat16)
    kb = k_ref[0]
    s = jax.lax.dot_general(qb, kb, (((1,), (1,)), ((), ())),
                            preferred_element_type=jnp.float32)
    uq = u_ref[0, :, pl.ds((nt - 1 - i) * BQ, SK)]
    bias = jnp.maximum(uq, vb_ref[0])
    row_id = i * BQ + jax.lax.broadcasted_iota(jnp.int32, (BQ, SK), 0)
    col_id = jax.lax.broadcasted_iota(jnp.int32, (BQ, SK), 1)
    sm = jnp.where(col_id <= row_id, s + bias, NEG)
    m = jnp.max(sm, axis=1, keepdims=True)
    p = jnp.exp(sm - m)
    l = jnp.sum(p, axis=1, keepdims=True)
    pn = (p * (1.0 / l)).astype(jnp.bfloat16)
    acc = jax.lax.dot_general(pn, v_ref[0],
                              (((1,), (0,)), ((), ())),
                              preferred_element_type=jnp.float32)
    o_ref[0] = acc


def _attn_bucket(qh, kb, vb16, v_bias_r, u_q, i0, ni, *, interpret=False):
    H, S, D = kb.shape
    nt = S // BQ
    sk = (i0 + ni) * BQ
    scale = 1.0 / np.sqrt(D)
    body = functools.partial(_attn_body, scale=scale, nt=nt, i0=i0)
    return pl.pallas_call(
        body,
        grid=(H, ni),
        in_specs=[
            pl.BlockSpec((1, BQ, D), lambda h, i: (h, i0 + i, 0)),
            pl.BlockSpec((1, sk, D), lambda h, i: (h, 0, 0)),
            pl.BlockSpec((1, sk, D), lambda h, i: (h, 0, 0)),
            pl.BlockSpec((1, 1, sk), lambda h, i: (h, 0, 0)),
            pl.BlockSpec((1, BQ, 2 * S), lambda h, i: (h, 0, 0)),
        ],
        out_specs=pl.BlockSpec((1, BQ, D), lambda h, i: (h, i, 0)),
        out_shape=jax.ShapeDtypeStruct((H, ni * BQ, D), jnp.float32),
        interpret=interpret,
    )(qh, kb, vb16, v_bias_r, u_q)


def kernel(q, k, v):
    B, H, S, D = q.shape
    v_idx, s_idx = _pattern_indices(q, k)

    s_r = jnp.arange(S)
    v_bias = jnp.where((v_idx[..., None] == s_r).any(axis=-2),
                       0.0, NEG).astype(jnp.bfloat16)
    s_bias = jnp.where((s_idx[..., None] == s_r).any(axis=-2),
                       0.0, NEG).astype(jnp.bfloat16)

    nt = S // BQ
    w2 = 2 * S + 2 * BQ
    neg_b = jnp.float32(NEG).astype(jnp.bfloat16)
    h_vec = jnp.concatenate(
        [s_bias[0][:, ::-1], jnp.full((H, w2 - S), neg_b, s_bias.dtype)], axis=1)
    m_all = jnp.broadcast_to(h_vec[:, None, :], (H, BQ, w2))
    skew = (m_all.reshape(H, BQ * w2)[:, :BQ * (w2 - 1)]
            .reshape(H, BQ, w2 - 1))
    u_q = skew[:, :, BQ - 1:BQ - 1 + 2 * S]

    qh = q[0]
    kb = k[0].astype(jnp.bfloat16)
    vb16 = v[0].astype(jnp.bfloat16)
    vbr = v_bias[0][:, None, :]
    lo = _attn_bucket(qh, kb, vb16, vbr, u_q, 0, nt // 2)
    hi = _attn_bucket(qh, kb, vb16, vbr, u_q, nt // 2, nt - nt // 2)
    out = jnp.concatenate([lo, hi], axis=1)[None]

    qmin = jnp.minimum(v_idx.min(-1), s_idx.min(-1))
    mean_v = jnp.mean(v, axis=2, keepdims=True)
    empty = s_r[None, None, :, None] < qmin[:, :, None, None]
    return jnp.where(empty, mean_v, out)
